# Initial kernel scaffold; baseline (speedup 1.0000x reference)
import jax
import jax.numpy as jnp
from jax.experimental import pallas as pl


def kernel(x, W1, b1, W2, b2):
    raise NotImplementedError("write your pallas kernel here")


# speedup vs baseline: 16.5213x; 16.5213x over previous
import functools

import jax
import jax.numpy as jnp
from jax import lax
from jax.experimental import pallas as pl
from jax.experimental.pallas import tpu as pltpu
from jax.experimental.pallas import tpu_sc as plsc

_RAD2 = 0.36
_NSAMPLE = 32
_LANES = 16
_C_OUT = 128


def _mlp_body(x_ref, w1_ref, b1_ref, w2_ref, b2_ref, out_ref):
    x = x_ref[...]
    h1 = jnp.dot(x, w1_ref[...], preferred_element_type=jnp.float32,
                 precision=lax.Precision.HIGHEST)
    h1 = jnp.maximum(h1 + b1_ref[...], 0.0)
    h2 = jnp.dot(h1, w2_ref[...], preferred_element_type=jnp.float32,
                 precision=lax.Precision.HIGHEST)
    out_ref[...] = jnp.maximum(h2 + b2_ref[...], 0.0)


def _mlp(xp, w1p, b1, w2t, b2):
    bn = xp.shape[0]
    tile = 1024
    grid = bn // tile
    return pl.pallas_call(
        _mlp_body,
        grid=(grid,),
        in_specs=[
            pl.BlockSpec((tile, xp.shape[1]), lambda i: (i, 0)),
            pl.BlockSpec(w1p.shape, lambda i: (0, 0)),
            pl.BlockSpec(b1.shape, lambda i: (0, 0)),
            pl.BlockSpec(w2t.shape, lambda i: (0, 0)),
            pl.BlockSpec(b2.shape, lambda i: (0, 0)),
        ],
        out_specs=pl.BlockSpec((tile, _C_OUT), lambda i: (i, 0)),
        out_shape=jax.ShapeDtypeStruct((bn, _C_OUT), jnp.float32),
    )(xp, w1p, b1, w2t, b2)


def _sc_ballmax(xs, ys, zs, h2, batch, n):
    info = plsc.get_sparse_core_info()
    nw = info.num_cores * info.num_subcores
    bn = batch * n
    nq = bn // nw
    mesh = plsc.VectorSubcoreMesh(core_axis_name="c", subcore_axis_name="s")

    @functools.partial(
        pl.kernel,
        mesh=mesh,
        compiler_params=pltpu.CompilerParams(needs_layout_passes=False),
        out_type=jax.ShapeDtypeStruct((bn, _C_OUT), jnp.float32),
        scratch_types=[
            pltpu.VMEM((n,), jnp.float32),
            pltpu.VMEM((n,), jnp.float32),
            pltpu.VMEM((n,), jnp.float32),
            pltpu.VMEM((64,), jnp.int32),
            pltpu.VMEM((_NSAMPLE,), jnp.int32),
            pltpu.VMEM((_NSAMPLE, _C_OUT), jnp.float32),
            pltpu.VMEM((nq, _C_OUT), jnp.float32),
            pltpu.SemaphoreType.DMA,
        ],
    )
    def k(xs_h, ys_h, zs_h, h2_h, out_h,
          xs_v, ys_v, zs_v, found, idx32, rows, outbuf, sem):
        wid = lax.axis_index("s") * info.num_cores + lax.axis_index("c")
        qg0 = wid * nq
        b = qg0 // n
        base = b * n
        pltpu.sync_copy(xs_h.at[pl.ds(base, n)], xs_v)
        pltpu.sync_copy(ys_h.at[pl.ds(base, n)], ys_v)
        pltpu.sync_copy(zs_h.at[pl.ds(base, n)], zs_v)

        def per_query(qi, carry):
            q = (qg0 - base) + qi
            qg = qg0 + qi
            qv = jnp.full((_LANES,), q, jnp.int32)
            xq = plsc.load_gather(xs_v, [qv])
            yq = plsc.load_gather(ys_v, [qv])
            zq = plsc.load_gather(zs_v, [qv])
            pad = jnp.full((_LANES,), qg, jnp.int32)
            found[pl.ds(0, _LANES)] = pad
            found[pl.ds(_LANES, _LANES)] = pad

            def cond(st):
                jb, cnt = st
                return jnp.logical_and(cnt < _NSAMPLE, jb < n)

            def body(st):
                jb, cnt = st
                jv = jb + lax.iota(jnp.int32, _LANES)
                dx = xs_v[pl.ds(jb, _LANES)] - xq
                dy = ys_v[pl.ds(jb, _LANES)] - yq
                dz = zs_v[pl.ds(jb, _LANES)] - zq
                sq = dx * dx + dy * dy + dz * dz
                m = sq <= _RAD2
                plsc.store_compressed(found.at[pl.ds(cnt, _LANES)],
                                      jv + base, mask=m)
                cnt = cnt + plsc.all_reduce_population_count(m)[0]
                return jb + _LANES, cnt

            lax.while_loop(cond, body, (jnp.int32(0), jnp.int32(0)))

            idx32[pl.ds(0, _LANES)] = found[pl.ds(0, _LANES)]
            idx32[pl.ds(_LANES, _LANES)] = found[pl.ds(_LANES, _LANES)]
            pltpu.async_copy(h2_h.at[idx32], rows, sem).wait()
            for c in range(_C_OUT // _LANES):
                acc = rows[0, pl.ds(c * _LANES, _LANES)]
                for r in range(1, _NSAMPLE):
                    acc = jnp.maximum(acc, rows[r, pl.ds(c * _LANES, _LANES)])
                outbuf[qi, pl.ds(c * _LANES, _LANES)] = acc
            return carry

        lax.fori_loop(0, nq, per_query, jnp.int32(0))
        pltpu.sync_copy(outbuf, out_h.at[pl.ds(qg0, nq)])

    return k(xs, ys, zs, h2)


def kernel(x, W1, b1, W2, b2):
    batch, n, _ = x.shape
    bn = batch * n
    xf = x.reshape(bn, 3)
    xp = jnp.pad(xf, ((0, 0), (0, 5)))
    w1p = jnp.pad(W1.T, ((0, 5), (0, 0)))
    h2 = _mlp(xp, w1p, b1.reshape(1, -1), W2.T, b2.reshape(1, -1))
    out_t = _sc_ballmax(xf[:, 0], xf[:, 1], xf[:, 2], h2, batch, n)
    return out_t.reshape(batch, n, _C_OUT).transpose(0, 2, 1)


# speedup vs baseline: 31.9152x; 1.9318x over previous
import functools

import jax
import jax.numpy as jnp
from jax import lax
from jax.experimental import pallas as pl
from jax.experimental.pallas import tpu as pltpu
from jax.experimental.pallas import tpu_sc as plsc

_RAD2 = 0.36
_NSAMPLE = 32
_LANES = 16
_C_OUT = 128


def _mlp_body(x_ref, w1_ref, b1_ref, w2_ref, b2_ref, out_ref):
    x = x_ref[...]
    h1 = jnp.dot(x, w1_ref[...], preferred_element_type=jnp.float32,
                 precision=lax.Precision.HIGHEST)
    h1 = jnp.maximum(h1 + b1_ref[...], 0.0)
    h2 = jnp.dot(h1, w2_ref[...], preferred_element_type=jnp.float32,
                 precision=lax.Precision.HIGHEST)
    out_ref[...] = jnp.maximum(h2 + b2_ref[...], 0.0)


def _mlp(xp, w1p, b1, w2t, b2):
    bn = xp.shape[0]
    tile = 1024
    grid = bn // tile
    return pl.pallas_call(
        _mlp_body,
        grid=(grid,),
        in_specs=[
            pl.BlockSpec((tile, xp.shape[1]), lambda i: (i, 0)),
            pl.BlockSpec(w1p.shape, lambda i: (0, 0)),
            pl.BlockSpec(b1.shape, lambda i: (0, 0)),
            pl.BlockSpec(w2t.shape, lambda i: (0, 0)),
            pl.BlockSpec(b2.shape, lambda i: (0, 0)),
        ],
        out_specs=pl.BlockSpec((tile, _C_OUT), lambda i: (i, 0)),
        out_shape=jax.ShapeDtypeStruct((bn, _C_OUT), jnp.float32),
    )(xp, w1p, b1, w2t, b2)


def _sc_ballmax(xs, ys, zs, h2, batch, n):
    info = plsc.get_sparse_core_info()
    nw = info.num_cores * info.num_subcores
    bn = batch * n
    nq = bn // nw
    grp = 4
    ngroups = nq // grp
    mesh = plsc.VectorSubcoreMesh(core_axis_name="c", subcore_axis_name="s")

    @functools.partial(
        pl.kernel,
        mesh=mesh,
        compiler_params=pltpu.CompilerParams(needs_layout_passes=False),
        out_type=jax.ShapeDtypeStruct((bn, _C_OUT), jnp.float32),
        scratch_types=[
            pltpu.VMEM((n,), jnp.float32),
            pltpu.VMEM((n,), jnp.float32),
            pltpu.VMEM((n,), jnp.float32),
            pltpu.VMEM((64,), jnp.int32),
            pltpu.VMEM((grp * _NSAMPLE,), jnp.int32),
            pltpu.VMEM((grp * _NSAMPLE,), jnp.int32),
            pltpu.VMEM((grp * _NSAMPLE, _C_OUT), jnp.float32),
            pltpu.VMEM((grp * _NSAMPLE, _C_OUT), jnp.float32),
            pltpu.VMEM((nq, _C_OUT), jnp.float32),
            pltpu.SemaphoreType.DMA,
            pltpu.SemaphoreType.DMA,
        ],
    )
    def k(xs_h, ys_h, zs_h, h2_h, out_h,
          xs_v, ys_v, zs_v, found, idx_a, idx_b, rows_a, rows_b,
          outbuf, sem_a, sem_b):
        wid = lax.axis_index("s") * info.num_cores + lax.axis_index("c")
        qg0 = wid * nq
        b = qg0 // n
        base = b * n
        pltpu.sync_copy(xs_h.at[pl.ds(base, n)], xs_v)
        pltpu.sync_copy(ys_h.at[pl.ds(base, n)], ys_v)
        pltpu.sync_copy(zs_h.at[pl.ds(base, n)], zs_v)

        def scan_query(qi, idx_ref, off):
            q = (qg0 - base) + qi
            qg = qg0 + qi
            qv = jnp.full((_LANES,), q, jnp.int32)
            xq = plsc.load_gather(xs_v, [qv])
            yq = plsc.load_gather(ys_v, [qv])
            zq = plsc.load_gather(zs_v, [qv])
            pad = jnp.full((_LANES,), qg, jnp.int32)
            found[pl.ds(0, _LANES)] = pad
            found[pl.ds(_LANES, _LANES)] = pad

            def cond(st):
                jb, cnt = st
                return jnp.logical_and(cnt < _NSAMPLE, jb < n)

            def body(st):
                jb, cnt = st
                jv = jb + lax.iota(jnp.int32, _LANES)
                dx = xs_v[pl.ds(jb, _LANES)] - xq
                dy = ys_v[pl.ds(jb, _LANES)] - yq
                dz = zs_v[pl.ds(jb, _LANES)] - zq
                sq = dx * dx + dy * dy + dz * dz
                m = sq <= _RAD2
                plsc.store_compressed(found.at[pl.ds(cnt, _LANES)],
                                      jv + base, mask=m)
                cnt = cnt + plsc.all_reduce_population_count(m)[0]
                return jb + _LANES, cnt

            lax.while_loop(cond, body, (jnp.int32(0), jnp.int32(0)))

            idx_ref[pl.ds(off, _LANES)] = found[pl.ds(0, _LANES)]
            idx_ref[pl.ds(off + _LANES, _LANES)] = found[pl.ds(_LANES, _LANES)]

        def scan_group(g, idx_ref):
            for t in range(grp):
                scan_query(g * grp + t, idx_ref, t * _NSAMPLE)

        def start(idx_ref, rows_ref, sem):
            pltpu.make_async_copy(h2_h.at[idx_ref], rows_ref, sem).start()

        def wait(idx_ref, rows_ref, sem):
            pltpu.make_async_copy(h2_h.at[idx_ref], rows_ref, sem).wait()

        def reduce_group(g, rows_ref):
            for t in range(grp):
                r0 = t * _NSAMPLE
                accs = tuple(rows_ref[r0, pl.ds(c * _LANES, _LANES)]
                             for c in range(_C_OUT // _LANES))
                def body(r, a):
                    return tuple(
                        jnp.maximum(a[c],
                                    rows_ref[r0 + r, pl.ds(c * _LANES, _LANES)])
                        for c in range(_C_OUT // _LANES))
                accs = lax.fori_loop(1, _NSAMPLE, body, accs)
                for c in range(_C_OUT // _LANES):
                    outbuf[g * grp + t, pl.ds(c * _LANES, _LANES)] = accs[c]

        scan_group(0, idx_a)
        start(idx_a, rows_a, sem_a)

        def pair(k_, carry):
            g = 2 * k_
            scan_group(g + 1, idx_b)
            start(idx_b, rows_b, sem_b)
            wait(idx_a, rows_a, sem_a)
            reduce_group(g, rows_a)
            scan_group(g + 2, idx_a)
            start(idx_a, rows_a, sem_a)
            wait(idx_b, rows_b, sem_b)
            reduce_group(g + 1, rows_b)
            return carry

        lax.fori_loop(0, ngroups // 2 - 1, pair, jnp.int32(0))

        gl = ngroups - 2
        scan_group(gl + 1, idx_b)
        start(idx_b, rows_b, sem_b)
        wait(idx_a, rows_a, sem_a)
        reduce_group(gl, rows_a)
        wait(idx_b, rows_b, sem_b)
        reduce_group(gl + 1, rows_b)

        pltpu.sync_copy(outbuf, out_h.at[pl.ds(qg0, nq)])

    return k(xs, ys, zs, h2)


def kernel(x, W1, b1, W2, b2):
    batch, n, _ = x.shape
    bn = batch * n
    xf = x.reshape(bn, 3)
    xp = jnp.pad(xf, ((0, 0), (0, 5)))
    w1p = jnp.pad(W1.T, ((0, 5), (0, 0)))
    h2 = _mlp(xp, w1p, b1.reshape(1, -1), W2.T, b2.reshape(1, -1))
    out_t = _sc_ballmax(xf[:, 0], xf[:, 1], xf[:, 2], h2, batch, n)
    return out_t.reshape(batch, n, _C_OUT).transpose(0, 2, 1)
